# baseline (speedup 1.0000x reference)
import functools

import jax
import jax.numpy as jnp
from jax import lax
from jax.experimental import pallas as pl
from jax.experimental.pallas import tpu as pltpu
from jax.experimental.pallas import tpu_sc as plsc

_NUM_CORES = 2
_NUM_SUBCORES = 16
_NUM_WORKERS = _NUM_CORES * _NUM_SUBCORES
_LANES = 16

_CHUNK = 128
_NBUF = 2


def _gather_rows(idx, table_w):
    n = idx.shape[0]
    d = 32
    per_w = n // _NUM_WORKERS
    n_chunks = per_w // _CHUNK
    mesh = plsc.VectorSubcoreMesh(
        core_axis_name="c", subcore_axis_name="s",
        num_cores=_NUM_CORES, num_subcores=_NUM_SUBCORES)

    @functools.partial(
        pl.kernel,
        out_type=jax.ShapeDtypeStruct((n, d), jnp.float32),
        mesh=mesh,
        scratch_types=[
            pltpu.VMEM((per_w,), jnp.int32),
            [pltpu.VMEM((_CHUNK,), jnp.int32)] * _NBUF,
            [pltpu.VMEM((_CHUNK, 128), jnp.float32)] * _NBUF,
            [pltpu.VMEM((_CHUNK, d), jnp.float32)] * _NBUF,
            [pltpu.SemaphoreType.DMA] * _NBUF,
            [pltpu.SemaphoreType.DMA] * _NBUF,
        ],
        compiler_params=pltpu.CompilerParams(needs_layout_passes=False),
    )
    def k(idx_hbm, table_hbm, out_hbm, idx_v, jv, big, outb, gsem, wsem):
        wid = lax.axis_index("s") * _NUM_CORES + lax.axis_index("c")
        base = wid * per_w
        pltpu.sync_copy(idx_hbm.at[pl.ds(base, per_w)], idx_v)

        def start_gather(c, b):
            @pl.loop(0, _CHUNK // _LANES)
            def _jloop(g):
                v = idx_v[pl.ds(c * _CHUNK + g * _LANES, _LANES)]
                jv[b][pl.ds(g * _LANES, _LANES)] = jax.lax.shift_right_logical(v, 2)
            pltpu.async_copy(table_hbm.at[jv[b]], big[b], gsem[b])

        for b in range(_NBUF):
            start_gather(b, b)

        @pl.loop(0, n_chunks, step=_NBUF)
        def _chunk_loop(c0):
            for b in range(_NBUF):
                c = c0 + b
                pltpu.make_async_copy(
                    table_hbm.at[jv[b]], big[b], gsem[b]).wait()

                @pl.loop(0, _CHUNK // _LANES)
                def _sel(g):
                    iv = lax.iota(jnp.int32, _LANES)
                    q = idx_v[pl.ds(c * _CHUNK + g * _LANES, _LANES)] & 3
                    colbase = q * d
                    rowv = iv + g * _LANES
                    for kk in range(d):
                        vals = plsc.load_gather(big[b], [rowv, colbase + kk])
                        plsc.store_scatter(
                            outb[b], [rowv, jnp.full((_LANES,), kk, jnp.int32)],
                            vals)

                pltpu.async_copy(
                    outb[b], out_hbm.at[pl.ds(base + c * _CHUNK, _CHUNK)],
                    wsem[b])
                pltpu.make_async_copy(
                    outb[b], out_hbm.at[pl.ds(base + c * _CHUNK, _CHUNK)],
                    wsem[b]).wait()

                @pl.when(c + _NBUF < n_chunks)
                def _():
                    start_gather(c + _NBUF, b)

    return k(idx, table_w)


def kernel(key, table):
    idx = key.reshape(-1)
    table_w = table.reshape(250000, 128)
    out = _gather_rows(idx, table_w)
    return out.reshape(key.shape + (table.shape[1],))


# speedup vs baseline: 1.2941x; 1.2941x over previous
import functools

import jax
import jax.numpy as jnp
from jax import lax
from jax.experimental import pallas as pl
from jax.experimental.pallas import tpu as pltpu
from jax.experimental.pallas import tpu_sc as plsc

_NUM_CORES = 2
_NUM_SUBCORES = 16
_NUM_WORKERS = _NUM_CORES * _NUM_SUBCORES

_COLS = 2048

_CHUNK = 640
_NBUF = 2


def _tc_pack(tt):
    n = tt.shape[1]
    grid = (n + _COLS - 1) // _COLS

    def body(tt_ref, out_ref):
        xt = jnp.transpose(tt_ref[...], (1, 0))
        x3 = xt.reshape(_COLS // 4, 4, 32)
        for a in range(4):
            out_ref[:, 32 * a:32 * (a + 1)] = x3[:, a, :]

    return pl.pallas_call(
        body,
        grid=(grid,),
        in_specs=[pl.BlockSpec((32, _COLS), lambda g: (0, g))],
        out_specs=pl.BlockSpec((_COLS // 4, 128), lambda g: (g, 0)),
        out_shape=jax.ShapeDtypeStruct((n // 4, 128), jnp.float32),
    )(tt)


def _sc_gather(idx, table_lin):
    n = idx.shape[0]
    d = table_lin.shape[1]
    per_w = n // _NUM_WORKERS
    n_chunks = per_w // _CHUNK
    mesh = plsc.VectorSubcoreMesh(
        core_axis_name="c", subcore_axis_name="s",
        num_cores=_NUM_CORES, num_subcores=_NUM_SUBCORES)

    @functools.partial(
        pl.kernel,
        out_type=jax.ShapeDtypeStruct((n, d), jnp.float32),
        mesh=mesh,
        scratch_types=[
            pltpu.VMEM((per_w,), jnp.int32),
            [pltpu.VMEM((_CHUNK, d), jnp.float32)] * _NBUF,
            [pltpu.SemaphoreType.DMA] * _NBUF,
            [pltpu.SemaphoreType.DMA] * _NBUF,
        ],
        compiler_params=pltpu.CompilerParams(use_tc_tiling_on_sc=False),
    )
    def k(idx_hbm, table_hbm, out_hbm, idx_v, rows, gsem, wsem):
        wid = lax.axis_index("s") * _NUM_CORES + lax.axis_index("c")
        base = wid * per_w
        pltpu.sync_copy(idx_hbm.at[pl.ds(base, per_w)], idx_v)

        def start_gather(c, b):
            pltpu.async_copy(
                table_hbm.at[idx_v.at[pl.ds(c * _CHUNK, _CHUNK)]],
                rows[b], gsem[b])

        for b in range(_NBUF):
            start_gather(b, b)

        @pl.loop(0, n_chunks, step=_NBUF)
        def _chunk_loop(c0):
            for b in range(_NBUF):
                c = c0 + b
                pltpu.make_async_copy(
                    table_hbm.at[idx_v.at[pl.ds(c * _CHUNK, _CHUNK)]],
                    rows[b], gsem[b]).wait()
                dst = out_hbm.at[pl.ds(base + c * _CHUNK, _CHUNK)]
                pltpu.async_copy(rows[b], dst, wsem[b])
                pltpu.make_async_copy(rows[b], dst, wsem[b]).wait()

                @pl.when(c + _NBUF < n_chunks)
                def _():
                    start_gather(c + _NBUF, b)

    return k(idx, table_lin)


def kernel(key, table):
    idx = key.reshape(-1)
    table_w = _tc_pack(table.T)
    table_lin = table_w.reshape(table.shape)
    out = _sc_gather(idx, table_lin)
    return out.reshape(key.shape + (table.shape[1],))
